# baseline (speedup 1.0000x reference)
import functools

import numpy as np
import jax
import jax.numpy as jnp
from jax.experimental import pallas as pl
from jax.experimental.pallas import tpu as pltpu

_VMEM_LIMIT_BYTES = 100 * 1024 * 1024
_C = 64
_F = 4
_H = 32
_LO = 16
_GUARD = 8
_MAX_CHUNK = 2048


def _conv_pattern(K, dilation, fold_in, stride=1, full_span=False):
    fold_out = fold_in // (2 if stride == 2 else 1)
    hits = {}
    for j in range(fold_out):
        for k in range(K):
            u = 2 * j - 1 + k if stride == 2 else j + (k - K // 2) * dilation
            hits.setdefault(u // fold_in, []).append((k, u % fold_in, j))
    rs = sorted(hits)
    P = np.zeros((K, len(rs), fold_in, fold_out), np.float32)
    spans = []
    for ri, r in enumerate(rs):
        us = [u for _, u, _ in hits[r]]
        spans.append((r, 0, fold_in) if full_span
                     else (r, min(us), max(us) + 1))
        for k, u, j in hits[r]:
            P[k, ri, u, j] = 1.0
    return P, spans


def _stage_arrays(down_w, down_b, res_params, dilations, final_wb,
                  raw_input=False):
    pd, spans_d = _conv_pattern(4, 1, 2 * _F, stride=2)
    down_mats = jnp.einsum('kruj,kio->ruijo', pd, down_w)
    down_mats = down_mats.reshape(3, 2 * _F * _C, _F * _C)
    down_mats = down_mats.astype(jnp.float32 if raw_input else jnp.bfloat16)
    down_bias = _tile_bias(down_b[None])

    conv_ws = [w3 for w3, _, _, _ in res_params]
    conv_bs = [b3 for _, b3, _, _ in res_params]
    dils = list(dilations)
    if final_wb is not None:
        conv_ws.append(final_wb[0])
        conv_bs.append(final_wb[1])
        dils.append(1)
    pats, all_spans = [], []
    for d in dils:
        p, spans = _conv_pattern(3, d, _F)
        pats.append(p)
        all_spans.append(tuple(spans))
    pc = np.stack(pats)
    conv_mats = jnp.einsum('ckruj,ckio->cruijo', pc, jnp.stack(conv_ws))
    conv_mats = conv_mats.reshape(len(dils), 3, _F * _C,
                                  _F * _C).astype(jnp.bfloat16)
    conv_biases = _tile_bias(jnp.stack(conv_bs))

    eye = np.eye(_F, dtype=np.float32)
    w1_mats = jnp.einsum('uj,cio->cuijo',
                         eye, jnp.stack([w1[0] for _, _, w1, _ in res_params]))
    w1_mats = w1_mats.reshape(4, _F * _C, _F * _C).astype(jnp.bfloat16)
    b1_biases = _tile_bias(jnp.stack([b1 for _, _, _, b1 in res_params]))

    ops = [('down', None, tuple(spans_d))]
    for ci in range(4):
        ops.append(('res', ci, all_spans[ci]))
    if final_wb is not None:
        ops.append(('final', 4, all_spans[4]))
    arrays = [down_mats, down_bias, conv_mats, conv_biases, w1_mats, b1_biases]
    return arrays, ops


def _tile_bias(b):
    return jnp.tile(b.astype(jnp.float32), (1, _F))


def _chain_kernel(x_hbm, down_mats, down_bias, conv_mats, conv_biases,
                  w1_mats, b1_biases, o_ref, wins, buf0, buf1, sem, *,
                  ops, chunk, n_chunks, n_batch, lanes_in, out_f32, unfold):
    n = pl.program_id(0)
    c = pl.program_id(1)
    g = n * n_chunks + c
    slot = jax.lax.rem(g, 2)
    wrows = chunk + 2 * _H
    lanes_mid = _F * _C
    cin = _C
    u = unfold
    lanes_win = lanes_in // u

    def dma_descs(nn, cc, sl):
        c0 = cc * chunk
        mid = pltpu.make_async_copy(
            x_hbm.at[nn, pl.ds(c0 * u, chunk * u), :],
            wins.at[sl, pl.ds(_H * u, chunk * u), :], sem.at[sl, 0])
        top = pltpu.make_async_copy(
            x_hbm.at[nn, pl.ds((c0 - _H) * u, _H * u), :],
            wins.at[sl, pl.ds(0, _H * u), :], sem.at[sl, 1])
        bot = pltpu.make_async_copy(
            x_hbm.at[nn, pl.ds((c0 + chunk) * u, _H * u), :],
            wins.at[sl, pl.ds((_H + chunk) * u, _H * u), :], sem.at[sl, 2])
        return mid, top, bot

    def issue(nn, cc, sl):
        mid, top, bot = dma_descs(nn, cc, sl)
        mid.start()

        @pl.when(cc != 0)
        def _():
            top.start()

        @pl.when(cc != n_chunks - 1)
        def _():
            bot.start()

    @pl.when(g == 0)
    def _():
        issue(n, c, slot)

    @pl.when(g + 1 < n_batch * n_chunks)
    def _():
        g1 = g + 1
        issue(g1 // n_chunks, jax.lax.rem(g1, n_chunks), 1 - slot)

    mid, top, bot = dma_descs(n, c, slot)
    mid.wait()

    @pl.when(c != 0)
    def _():
        top.wait()

    @pl.when(c == 0)
    def _():
        wins[slot, 0:_H * u, :] = jnp.zeros((_H * u, lanes_win), wins.dtype)

    @pl.when(c != n_chunks - 1)
    def _():
        bot.wait()

    @pl.when(c == n_chunks - 1)
    def _():
        wins[slot, (_H + chunk) * u:wrows * u, :] = jnp.zeros(
            (_H * u, lanes_win), wins.dtype)

    win = wins.at[slot]
    lo, hi = _LO, wrows - _LO

    for b in (buf0, buf1):
        b[lo - _GUARD:lo, :] = jnp.zeros((_GUARD, lanes_mid), b.dtype)
        b[hi:hi + _GUARD, :] = jnp.zeros((_GUARD, lanes_mid), b.dtype)

    def conv_acc(src, spans, get_mat, bias, relu_in):
        acc = None
        for ti, (r, u_lo, u_hi) in enumerate(spans):
            sl = src[lo + r:hi + r, u_lo * cin:u_hi * cin]
            if relu_in:
                sl = jnp.maximum(sl, jnp.zeros((), sl.dtype))
            part = jnp.dot(sl, get_mat(ti, u_lo, u_hi),
                           preferred_element_type=jnp.float32)
            acc = part if acc is None else acc + part
        return acc + bias.astype(jnp.float32)

    def edge_zero(dst):
        @pl.when(c == 0)
        def _():
            dst[0:_H, :] = jnp.zeros((_H, lanes_mid), dst.dtype)

        @pl.when(c == n_chunks - 1)
        def _():
            dst[_H + chunk:wrows, :] = jnp.zeros((_H, lanes_mid), dst.dtype)

    src = win
    bufs = [buf0, buf1]
    bi = 0
    for kind, ci, spans in ops:
        if kind == 'down':
            acc = conv_acc(
                src, spans,
                lambda ti, a, b: down_mats[ti, a * cin:b * cin, :],
                down_bias[0:1, :], False)
            dst = bufs[bi]
            bi ^= 1
            dst[lo:hi, :] = acc.astype(dst.dtype)
            edge_zero(dst)
            src = dst
        elif kind == 'res':
            acc = conv_acc(
                src, spans,
                lambda ti, a, b: conv_mats[ci, ti, a * cin:b * cin, :],
                conv_biases[ci:ci + 1, :], True)
            h2 = jnp.maximum(acc, 0.0).astype(jnp.bfloat16)
            y = jnp.dot(h2, w1_mats[ci], preferred_element_type=jnp.float32)
            y = y + b1_biases[ci:ci + 1, :].astype(jnp.float32)
            out = src[lo:hi, :].astype(jnp.float32) + y
            dst = bufs[bi]
            bi ^= 1
            dst[lo:hi, :] = out.astype(dst.dtype)
            edge_zero(dst)
            src = dst
        else:
            acc = conv_acc(
                src, spans,
                lambda ti, a, b: conv_mats[ci, ti, a * cin:b * cin, :],
                conv_biases[ci:ci + 1, :], False)
            out = acc[_H - lo:_H - lo + chunk, :]
            o_ref[0] = out if out_f32 else out.astype(o_ref.dtype)
            return
    o_ref[0] = src[_H:_H + chunk, :]


def _run_stage(x_in, arrays, ops, n_rows_out, out_dtype, unfold=1):
    n_batch = x_in.shape[0]
    lanes_win = x_in.shape[2]
    lanes_in = lanes_win * unfold
    chunk = min(_MAX_CHUNK, n_rows_out)
    n_chunks = n_rows_out // chunk
    wrows = chunk + 2 * _H
    out_f32 = out_dtype == jnp.float32
    body = functools.partial(_chain_kernel, ops=ops, chunk=chunk,
                             n_chunks=n_chunks, n_batch=n_batch,
                             lanes_in=lanes_in, out_f32=out_f32,
                             unfold=unfold)
    in_specs = [pl.BlockSpec(memory_space=pl.ANY)]
    for w in arrays:
        in_specs.append(
            pl.BlockSpec(w.shape, lambda *_, nd=w.ndim: (0,) * nd))
    out_shape = jax.ShapeDtypeStruct((n_batch, n_rows_out, _F * _C),
                                     out_dtype)
    out_spec = pl.BlockSpec((1, chunk, _F * _C), lambda n, c: (n, c, 0))
    return pl.pallas_call(
        body,
        out_shape=out_shape,
        grid=(n_batch, n_chunks),
        in_specs=in_specs,
        out_specs=out_spec,
        scratch_shapes=[pltpu.VMEM((2, wrows * unfold, lanes_win), x_in.dtype),
                        pltpu.VMEM((wrows, _F * _C), jnp.bfloat16),
                        pltpu.VMEM((wrows, _F * _C), jnp.bfloat16),
                        pltpu.SemaphoreType.DMA((2, 3))],
        compiler_params=pltpu.CompilerParams(
            dimension_semantics=("arbitrary", "arbitrary"),
            vmem_limit_bytes=_VMEM_LIMIT_BYTES),
    )(x_in, *arrays)


def kernel(x, down_0_w, down_0_b, res_0_0_w3, res_0_0_b3, res_0_0_w1, res_0_0_b1, res_0_1_w3, res_0_1_b3, res_0_1_w1, res_0_1_b1, res_0_2_w3, res_0_2_b3, res_0_2_w1, res_0_2_b1, res_0_3_w3, res_0_3_b3, res_0_3_w1, res_0_3_b1, down_1_w, down_1_b, res_1_0_w3, res_1_0_b3, res_1_0_w1, res_1_0_b1, res_1_1_w3, res_1_1_b3, res_1_1_w1, res_1_1_b1, res_1_2_w3, res_1_2_b3, res_1_2_w1, res_1_2_b1, res_1_3_w3, res_1_3_b3, res_1_3_w1, res_1_3_b1, out_w, out_b):
    n_batch, t_len, c_in = x.shape
    dilations = (1, 2, 4, 8)

    arrs_a, ops_a = _stage_arrays(
        down_0_w, down_0_b,
        [(res_0_0_w3, res_0_0_b3, res_0_0_w1, res_0_0_b1),
         (res_0_1_w3, res_0_1_b3, res_0_1_w1, res_0_1_b1),
         (res_0_2_w3, res_0_2_b3, res_0_2_w1, res_0_2_b1),
         (res_0_3_w3, res_0_3_b3, res_0_3_w1, res_0_3_b1)], dilations, None,
        raw_input=True)
    x_folded = x.reshape(n_batch, t_len // (2 * _F), 2 * _F * c_in)
    h = _run_stage(x_folded, arrs_a, ops_a, t_len // (2 * _F), jnp.bfloat16)

    h_folded = h.reshape(n_batch, t_len // (4 * _F), 2 * _F * _C)
    arrs_b, ops_b = _stage_arrays(
        down_1_w, down_1_b,
        [(res_1_0_w3, res_1_0_b3, res_1_0_w1, res_1_0_b1),
         (res_1_1_w3, res_1_1_b3, res_1_1_w1, res_1_1_b1),
         (res_1_2_w3, res_1_2_b3, res_1_2_w1, res_1_2_b1),
         (res_1_3_w3, res_1_3_b3, res_1_3_w1, res_1_3_b1)], dilations,
        (out_w, out_b))
    out = _run_stage(h_folded, arrs_b, ops_b, t_len // (4 * _F), jnp.float32)
    return out.reshape(n_batch, t_len // 4, _C)


# speedup vs baseline: 1.0185x; 1.0185x over previous
import functools

import numpy as np
import jax
import jax.numpy as jnp
from jax.experimental import pallas as pl
from jax.experimental.pallas import tpu as pltpu

_VMEM_LIMIT_BYTES = 100 * 1024 * 1024
_C = 64
_F = 4
_H = 32
_LO = 16
_GUARD = 8
_MAX_CHUNK = 2048


def _conv_pattern(K, dilation, fold_in, stride=1, full_span=False):
    fold_out = fold_in // (2 if stride == 2 else 1)
    hits = {}
    for j in range(fold_out):
        for k in range(K):
            u = 2 * j - 1 + k if stride == 2 else j + (k - K // 2) * dilation
            hits.setdefault(u // fold_in, []).append((k, u % fold_in, j))
    rs = sorted(hits)
    P = np.zeros((K, len(rs), fold_in, fold_out), np.float32)
    spans = []
    for ri, r in enumerate(rs):
        us = [u for _, u, _ in hits[r]]
        spans.append((r, 0, fold_in) if full_span
                     else (r, min(us), max(us) + 1))
        for k, u, j in hits[r]:
            P[k, ri, u, j] = 1.0
    return P, spans


def _stage_arrays(down_w, down_b, res_params, dilations, final_wb,
                  raw_input=False):
    pd, spans_d = _conv_pattern(4, 1, 2 * _F, stride=2)
    down_mats = jnp.einsum('kruj,kio->ruijo', pd, down_w)
    down_mats = down_mats.reshape(3, 2 * _F * _C, _F * _C)
    down_mats = down_mats.astype(jnp.float32 if raw_input else jnp.bfloat16)
    down_bias = _tile_bias(down_b[None])

    conv_ws = [w3 for w3, _, _, _ in res_params]
    conv_bs = [b3 for _, b3, _, _ in res_params]
    dils = list(dilations)
    if final_wb is not None:
        conv_ws.append(final_wb[0])
        conv_bs.append(final_wb[1])
        dils.append(1)
    pats, all_spans = [], []
    for d in dils:
        p, spans = _conv_pattern(3, d, _F)
        pats.append(p)
        all_spans.append(tuple(spans))
    pc = np.stack(pats)
    conv_mats = jnp.einsum('ckruj,ckio->cruijo', pc, jnp.stack(conv_ws))
    conv_mats = conv_mats.reshape(len(dils), 3, _F * _C,
                                  _F * _C).astype(jnp.bfloat16)
    conv_biases = _tile_bias(jnp.stack(conv_bs))

    eye = np.eye(_F, dtype=np.float32)
    w1_mats = jnp.einsum('uj,cio->cuijo',
                         eye, jnp.stack([w1[0] for _, _, w1, _ in res_params]))
    w1_mats = w1_mats.reshape(4, _F * _C, _F * _C).astype(jnp.bfloat16)
    b1_biases = _tile_bias(jnp.stack([b1 for _, _, _, b1 in res_params]))

    ops = [('down', None, tuple(spans_d))]
    for ci in range(4):
        ops.append(('res', ci, all_spans[ci]))
    if final_wb is not None:
        ops.append(('final', 4, all_spans[4]))
    arrays = [down_mats, down_bias, conv_mats, conv_biases, w1_mats, b1_biases]
    return arrays, ops


def _tile_bias(b):
    return jnp.tile(b.astype(jnp.float32), (1, _F))


def _chain_kernel(x_hbm, down_mats, down_bias, conv_mats, conv_biases,
                  w1_mats, b1_biases, o_ref, wins, buf0, buf1, sem, *,
                  ops, chunk, n_chunks, n_batch, lanes_in, out_f32, dma_fold):
    n = pl.program_id(0)
    c = pl.program_id(1)
    g = n * n_chunks + c
    slot = jax.lax.rem(g, 2)
    wrows = chunk + 2 * _H
    lanes_mid = _F * _C
    cin = _C
    nu = lanes_in // _C

    def fold_descs(nn, cc, sl, lo_r, rows):
        c0 = cc * chunk
        return [pltpu.make_async_copy(
            x_hbm.at[nn, pl.ds(c0 - _H + lo_r, rows), ui, :],
            wins.at[sl, pl.ds(lo_r, rows), ui * _C:(ui + 1) * _C],
            sem.at[sl, ui]) for ui in range(nu)]

    def fold_cases(cc):
        if n_chunks == 1:
            return [(None, _H, chunk)]
        return [(cc == 0, _H, wrows - _H),
                (cc == n_chunks - 1, 0, wrows - _H),
                ((cc != 0) & (cc != n_chunks - 1), 0, wrows)]

    def fold_run(nn, cc, sl, fn):
        for guard, lo_r, rows in fold_cases(cc):
            if guard is None:
                for cp in fold_descs(nn, cc, sl, lo_r, rows):
                    fn(cp)
            else:
                @pl.when(guard)
                def _(lo_r=lo_r, rows=rows):
                    for cp in fold_descs(nn, cc, sl, lo_r, rows):
                        fn(cp)

    def flat_descs(nn, cc, sl):
        c0 = cc * chunk
        mid = pltpu.make_async_copy(
            x_hbm.at[nn, pl.ds(c0, chunk), :],
            wins.at[sl, pl.ds(_H, chunk), :], sem.at[sl, 0])
        top = pltpu.make_async_copy(
            x_hbm.at[nn, pl.ds(c0 - _H, _H), :],
            wins.at[sl, pl.ds(0, _H), :], sem.at[sl, 1])
        bot = pltpu.make_async_copy(
            x_hbm.at[nn, pl.ds(c0 + chunk, _H), :],
            wins.at[sl, pl.ds(_H + chunk, _H), :], sem.at[sl, 2])
        return mid, top, bot

    def issue(nn, cc, sl):
        if dma_fold:
            fold_run(nn, cc, sl, lambda cp: cp.start())
            return
        mid, top, bot = flat_descs(nn, cc, sl)
        mid.start()

        @pl.when(cc != 0)
        def _():
            top.start()

        @pl.when(cc != n_chunks - 1)
        def _():
            bot.start()

    @pl.when(g == 0)
    def _():
        issue(n, c, slot)

    @pl.when(g + 1 < n_batch * n_chunks)
    def _():
        g1 = g + 1
        issue(g1 // n_chunks, jax.lax.rem(g1, n_chunks), 1 - slot)

    if dma_fold:
        fold_run(n, c, slot, lambda cp: cp.wait())
    else:
        mid, top, bot = flat_descs(n, c, slot)
        mid.wait()

        @pl.when(c != 0)
        def _():
            top.wait()

        @pl.when(c != n_chunks - 1)
        def _():
            bot.wait()

    @pl.when(c == 0)
    def _():
        wins[slot, 0:_H, :] = jnp.zeros((_H, lanes_in), wins.dtype)

    @pl.when(c == n_chunks - 1)
    def _():
        wins[slot, _H + chunk:wrows, :] = jnp.zeros((_H, lanes_in), wins.dtype)

    win = wins.at[slot]
    lo, hi = _LO, wrows - _LO

    for b in (buf0, buf1):
        b[lo - _GUARD:lo, :] = jnp.zeros((_GUARD, lanes_mid), b.dtype)
        b[hi:hi + _GUARD, :] = jnp.zeros((_GUARD, lanes_mid), b.dtype)

    def conv_acc(src, spans, get_mat, bias, relu_in):
        acc = None
        for ti, (r, u_lo, u_hi) in enumerate(spans):
            sl = src[lo + r:hi + r, u_lo * cin:u_hi * cin]
            if relu_in:
                sl = jnp.maximum(sl, jnp.zeros((), sl.dtype))
            part = jnp.dot(sl, get_mat(ti, u_lo, u_hi),
                           preferred_element_type=jnp.float32)
            acc = part if acc is None else acc + part
        return acc + bias.astype(jnp.float32)

    def edge_zero(dst):
        @pl.when(c == 0)
        def _():
            dst[0:_H, :] = jnp.zeros((_H, lanes_mid), dst.dtype)

        @pl.when(c == n_chunks - 1)
        def _():
            dst[_H + chunk:wrows, :] = jnp.zeros((_H, lanes_mid), dst.dtype)

    src = win
    bufs = [buf0, buf1]
    bi = 0
    for kind, ci, spans in ops:
        if kind == 'down':
            acc = conv_acc(
                src, spans,
                lambda ti, a, b: down_mats[ti, a * cin:b * cin, :],
                down_bias[0:1, :], False)
            dst = bufs[bi]
            bi ^= 1
            dst[lo:hi, :] = acc.astype(dst.dtype)
            edge_zero(dst)
            src = dst
        elif kind == 'res':
            acc = conv_acc(
                src, spans,
                lambda ti, a, b: conv_mats[ci, ti, a * cin:b * cin, :],
                conv_biases[ci:ci + 1, :], True)
            h2 = jnp.maximum(acc, 0.0).astype(jnp.bfloat16)
            y = jnp.dot(h2, w1_mats[ci], preferred_element_type=jnp.float32)
            y = y + b1_biases[ci:ci + 1, :].astype(jnp.float32)
            out = src[lo:hi, :].astype(jnp.float32) + y
            dst = bufs[bi]
            bi ^= 1
            dst[lo:hi, :] = out.astype(dst.dtype)
            edge_zero(dst)
            src = dst
        else:
            acc = conv_acc(
                src, spans,
                lambda ti, a, b: conv_mats[ci, ti, a * cin:b * cin, :],
                conv_biases[ci:ci + 1, :], False)
            out = acc[_H - lo:_H - lo + chunk, :]
            o_ref[0] = out if out_f32 else out.astype(o_ref.dtype)
            return
    o_ref[0] = src[_H:_H + chunk, :]


def _run_stage(x_in, arrays, ops, n_rows_out, out_dtype, dma_fold=False):
    n_batch = x_in.shape[0]
    lanes_in = x_in.shape[2] * (_C if dma_fold else 1)
    chunk = min(_MAX_CHUNK, n_rows_out)
    n_chunks = n_rows_out // chunk
    wrows = chunk + 2 * _H
    out_f32 = out_dtype == jnp.float32
    body = functools.partial(_chain_kernel, ops=ops, chunk=chunk,
                             n_chunks=n_chunks, n_batch=n_batch,
                             lanes_in=lanes_in, out_f32=out_f32,
                             dma_fold=dma_fold)
    in_specs = [pl.BlockSpec(memory_space=pl.ANY)]
    for w in arrays:
        in_specs.append(
            pl.BlockSpec(w.shape, lambda *_, nd=w.ndim: (0,) * nd))
    out_shape = jax.ShapeDtypeStruct((n_batch, n_rows_out, _F * _C),
                                     out_dtype)
    out_spec = pl.BlockSpec((1, chunk, _F * _C), lambda n, c: (n, c, 0))
    return pl.pallas_call(
        body,
        out_shape=out_shape,
        grid=(n_batch, n_chunks),
        in_specs=in_specs,
        out_specs=out_spec,
        scratch_shapes=[pltpu.VMEM((2, wrows, lanes_in), x_in.dtype),
                        pltpu.VMEM((wrows, _F * _C), jnp.bfloat16),
                        pltpu.VMEM((wrows, _F * _C), jnp.bfloat16),
                        pltpu.SemaphoreType.DMA((2, 8 if dma_fold else 3))],
        compiler_params=pltpu.CompilerParams(
            dimension_semantics=("arbitrary", "arbitrary"),
            vmem_limit_bytes=_VMEM_LIMIT_BYTES),
    )(x_in, *arrays)


def kernel(x, down_0_w, down_0_b, res_0_0_w3, res_0_0_b3, res_0_0_w1, res_0_0_b1, res_0_1_w3, res_0_1_b3, res_0_1_w1, res_0_1_b1, res_0_2_w3, res_0_2_b3, res_0_2_w1, res_0_2_b1, res_0_3_w3, res_0_3_b3, res_0_3_w1, res_0_3_b1, down_1_w, down_1_b, res_1_0_w3, res_1_0_b3, res_1_0_w1, res_1_0_b1, res_1_1_w3, res_1_1_b3, res_1_1_w1, res_1_1_b1, res_1_2_w3, res_1_2_b3, res_1_2_w1, res_1_2_b1, res_1_3_w3, res_1_3_b3, res_1_3_w1, res_1_3_b1, out_w, out_b):
    n_batch, t_len, c_in = x.shape
    dilations = (1, 2, 4, 8)

    arrs_a, ops_a = _stage_arrays(
        down_0_w, down_0_b,
        [(res_0_0_w3, res_0_0_b3, res_0_0_w1, res_0_0_b1),
         (res_0_1_w3, res_0_1_b3, res_0_1_w1, res_0_1_b1),
         (res_0_2_w3, res_0_2_b3, res_0_2_w1, res_0_2_b1),
         (res_0_3_w3, res_0_3_b3, res_0_3_w1, res_0_3_b1)], dilations, None,
        raw_input=False)
    x_folded = x.reshape(n_batch, t_len // (2 * _F),
                         2 * _F * c_in).astype(jnp.bfloat16)
    h = _run_stage(x_folded, arrs_a, ops_a, t_len // (2 * _F), jnp.bfloat16)

    h_folded = h.reshape(n_batch, t_len // (4 * _F), 2 * _F * _C)
    arrs_b, ops_b = _stage_arrays(
        down_1_w, down_1_b,
        [(res_1_0_w3, res_1_0_b3, res_1_0_w1, res_1_0_b1),
         (res_1_1_w3, res_1_1_b3, res_1_1_w1, res_1_1_b1),
         (res_1_2_w3, res_1_2_b3, res_1_2_w1, res_1_2_b1),
         (res_1_3_w3, res_1_3_b3, res_1_3_w1, res_1_3_b1)], dilations,
        (out_w, out_b))
    out = _run_stage(h_folded, arrs_b, ops_b, t_len // (4 * _F), jnp.float32)
    return out.reshape(n_batch, t_len // 4, _C)


# speedup vs baseline: 1.0197x; 1.0012x over previous
import functools

import numpy as np
import jax
import jax.numpy as jnp
from jax.experimental import pallas as pl
from jax.experimental.pallas import tpu as pltpu

_VMEM_LIMIT_BYTES = 100 * 1024 * 1024
_C = 64
_F = 4
_H = 32
_LO = 16
_GUARD = 8
_MAX_CHUNK = 2048


def _conv_pattern(K, dilation, fold_in, stride=1):
    fold_out = fold_in // (2 if stride == 2 else 1)
    hits = {}
    for j in range(fold_out):
        for k in range(K):
            u = 2 * j - 1 + k if stride == 2 else j + (k - K // 2) * dilation
            hits.setdefault(u // fold_in, []).append((k, u % fold_in, j))
    rs = sorted(hits)
    P = np.zeros((K, len(rs), fold_in, fold_out), np.float32)
    spans = []
    for ri, r in enumerate(rs):
        us = [u for _, u, _ in hits[r]]
        spans.append((r, min(us), max(us) + 1))
        for k, u, j in hits[r]:
            P[k, ri, u, j] = 1.0
    return P, spans


def _stage_arrays(down_w, down_b, res_params, dilations, final_wb):
    pd, spans_d = _conv_pattern(4, 1, 2 * _F, stride=2)
    down_mats = jnp.einsum('kruj,kio->ruijo', pd, down_w)
    down_mats = down_mats.reshape(3, 2 * _F * _C, _F * _C).astype(jnp.bfloat16)
    down_bias = _tile_bias(down_b[None])

    conv_ws = [w3 for w3, _, _, _ in res_params]
    conv_bs = [b3 for _, b3, _, _ in res_params]
    dils = list(dilations)
    if final_wb is not None:
        conv_ws.append(final_wb[0])
        conv_bs.append(final_wb[1])
        dils.append(1)
    pats, all_spans = [], []
    for d in dils:
        p, spans = _conv_pattern(3, d, _F)
        pats.append(p)
        all_spans.append(tuple(spans))
    pc = np.stack(pats)
    conv_mats = jnp.einsum('ckruj,ckio->cruijo', pc, jnp.stack(conv_ws))
    conv_mats = conv_mats.reshape(len(dils), 3, _F * _C,
                                  _F * _C).astype(jnp.bfloat16)
    conv_biases = _tile_bias(jnp.stack(conv_bs))

    eye = np.eye(_F, dtype=np.float32)
    w1_mats = jnp.einsum('uj,cio->cuijo',
                         eye, jnp.stack([w1[0] for _, _, w1, _ in res_params]))
    w1_mats = w1_mats.reshape(4, _F * _C, _F * _C).astype(jnp.bfloat16)
    b1_biases = _tile_bias(jnp.stack([b1 for _, _, _, b1 in res_params]))

    ops = [('down', None, tuple(spans_d))]
    for ci in range(4):
        ops.append(('res', ci, all_spans[ci]))
    if final_wb is not None:
        ops.append(('final', 4, all_spans[4]))
    arrays = [down_mats, down_bias, conv_mats, conv_biases, w1_mats, b1_biases]
    return arrays, ops


def _tile_bias(b):
    return jnp.tile(b.astype(jnp.float32), (1, _F))


def _chain_kernel(x_hbm, down_mats, down_bias, conv_mats, conv_biases,
                  w1_mats, b1_biases, o_ref, wins, buf0, buf1, sem, *,
                  ops, chunk, n_chunks, n_batch, lanes_in, out_f32):
    n = pl.program_id(0)
    c = pl.program_id(1)
    g = n * n_chunks + c
    slot = jax.lax.rem(g, 2)
    wrows = chunk + 2 * _H
    lanes_mid = _F * _C
    cin = _C

    def flat_descs(nn, cc, sl):
        c0 = cc * chunk
        mid = pltpu.make_async_copy(
            x_hbm.at[nn, pl.ds(c0, chunk), :],
            wins.at[sl, pl.ds(_H, chunk), :], sem.at[sl, 0])
        top = pltpu.make_async_copy(
            x_hbm.at[nn, pl.ds(c0 - _H, _H), :],
            wins.at[sl, pl.ds(0, _H), :], sem.at[sl, 1])
        bot = pltpu.make_async_copy(
            x_hbm.at[nn, pl.ds(c0 + chunk, _H), :],
            wins.at[sl, pl.ds(_H + chunk, _H), :], sem.at[sl, 2])
        return mid, top, bot

    def issue(nn, cc, sl):
        mid, top, bot = flat_descs(nn, cc, sl)
        mid.start()

        @pl.when(cc != 0)
        def _():
            top.start()

        @pl.when(cc != n_chunks - 1)
        def _():
            bot.start()

    @pl.when(g == 0)
    def _():
        issue(n, c, slot)

    @pl.when(g + 1 < n_batch * n_chunks)
    def _():
        g1 = g + 1
        issue(g1 // n_chunks, jax.lax.rem(g1, n_chunks), 1 - slot)

    mid, top, bot = flat_descs(n, c, slot)
    mid.wait()

    @pl.when(c != 0)
    def _():
        top.wait()

    @pl.when(c != n_chunks - 1)
    def _():
        bot.wait()

    @pl.when(c == 0)
    def _():
        wins[slot, 0:_H, :] = jnp.zeros((_H, lanes_in), wins.dtype)

    @pl.when(c == n_chunks - 1)
    def _():
        wins[slot, _H + chunk:wrows, :] = jnp.zeros((_H, lanes_in), wins.dtype)

    win = wins.at[slot]
    lo, hi = _LO, wrows - _LO

    for b in (buf0, buf1):
        b[lo - _GUARD:lo, :] = jnp.zeros((_GUARD, lanes_mid), b.dtype)
        b[hi:hi + _GUARD, :] = jnp.zeros((_GUARD, lanes_mid), b.dtype)

    def conv_acc(src, spans, get_mat, bias, relu_in):
        acc = None
        for ti, (r, u_lo, u_hi) in enumerate(spans):
            sl = src[lo + r:hi + r, u_lo * cin:u_hi * cin]
            if relu_in:
                sl = jnp.maximum(sl, jnp.zeros((), sl.dtype))
            part = jnp.dot(sl, get_mat(ti, u_lo, u_hi),
                           preferred_element_type=jnp.float32)
            acc = part if acc is None else acc + part
        return acc + bias.astype(jnp.float32)

    def edge_zero(dst):
        @pl.when(c == 0)
        def _():
            dst[0:_H, :] = jnp.zeros((_H, lanes_mid), dst.dtype)

        @pl.when(c == n_chunks - 1)
        def _():
            dst[_H + chunk:wrows, :] = jnp.zeros((_H, lanes_mid), dst.dtype)

    src = win
    bufs = [buf0, buf1]
    bi = 0
    for kind, ci, spans in ops:
        if kind == 'down':
            acc = conv_acc(
                src, spans,
                lambda ti, a, b: down_mats[ti, a * cin:b * cin, :],
                down_bias[0:1, :], False)
            dst = bufs[bi]
            bi ^= 1
            dst[lo:hi, :] = acc.astype(dst.dtype)
            edge_zero(dst)
            src = dst
        elif kind == 'res':
            acc = conv_acc(
                src, spans,
                lambda ti, a, b: conv_mats[ci, ti, a * cin:b * cin, :],
                conv_biases[ci:ci + 1, :], True)
            h2 = jnp.maximum(acc, 0.0).astype(jnp.bfloat16)
            y = jnp.dot(h2, w1_mats[ci], preferred_element_type=jnp.float32)
            y = y + b1_biases[ci:ci + 1, :].astype(jnp.float32)
            out = src[lo:hi, :].astype(jnp.float32) + y
            dst = bufs[bi]
            bi ^= 1
            dst[lo:hi, :] = out.astype(dst.dtype)
            edge_zero(dst)
            src = dst
        else:
            acc = conv_acc(
                src, spans,
                lambda ti, a, b: conv_mats[ci, ti, a * cin:b * cin, :],
                conv_biases[ci:ci + 1, :], False)
            out = acc[_H - lo:_H - lo + chunk, :]
            o_ref[0] = out if out_f32 else out.astype(o_ref.dtype)
            return
    o_ref[0] = src[_H:_H + chunk, :]


def _run_stage(x_in, arrays, ops, n_rows_out, out_dtype):
    n_batch = x_in.shape[0]
    lanes_in = x_in.shape[2]
    chunk = min(_MAX_CHUNK, n_rows_out)
    n_chunks = n_rows_out // chunk
    wrows = chunk + 2 * _H
    out_f32 = out_dtype == jnp.float32
    body = functools.partial(_chain_kernel, ops=ops, chunk=chunk,
                             n_chunks=n_chunks, n_batch=n_batch,
                             lanes_in=lanes_in, out_f32=out_f32)
    in_specs = [pl.BlockSpec(memory_space=pl.ANY)]
    for w in arrays:
        in_specs.append(
            pl.BlockSpec(w.shape, lambda *_, nd=w.ndim: (0,) * nd))
    out_shape = jax.ShapeDtypeStruct((n_batch, n_rows_out, _F * _C),
                                     out_dtype)
    out_spec = pl.BlockSpec((1, chunk, _F * _C), lambda n, c: (n, c, 0))
    return pl.pallas_call(
        body,
        out_shape=out_shape,
        grid=(n_batch, n_chunks),
        in_specs=in_specs,
        out_specs=out_spec,
        scratch_shapes=[pltpu.VMEM((2, wrows, lanes_in), x_in.dtype),
                        pltpu.VMEM((wrows, _F * _C), jnp.bfloat16),
                        pltpu.VMEM((wrows, _F * _C), jnp.bfloat16),
                        pltpu.SemaphoreType.DMA((2, 3))],
        compiler_params=pltpu.CompilerParams(
            dimension_semantics=("arbitrary", "arbitrary"),
            vmem_limit_bytes=_VMEM_LIMIT_BYTES),
    )(x_in, *arrays)


def kernel(x, down_0_w, down_0_b, res_0_0_w3, res_0_0_b3, res_0_0_w1, res_0_0_b1, res_0_1_w3, res_0_1_b3, res_0_1_w1, res_0_1_b1, res_0_2_w3, res_0_2_b3, res_0_2_w1, res_0_2_b1, res_0_3_w3, res_0_3_b3, res_0_3_w1, res_0_3_b1, down_1_w, down_1_b, res_1_0_w3, res_1_0_b3, res_1_0_w1, res_1_0_b1, res_1_1_w3, res_1_1_b3, res_1_1_w1, res_1_1_b1, res_1_2_w3, res_1_2_b3, res_1_2_w1, res_1_2_b1, res_1_3_w3, res_1_3_b3, res_1_3_w1, res_1_3_b1, out_w, out_b):
    n_batch, t_len, c_in = x.shape
    dilations = (1, 2, 4, 8)

    arrs_a, ops_a = _stage_arrays(
        down_0_w, down_0_b,
        [(res_0_0_w3, res_0_0_b3, res_0_0_w1, res_0_0_b1),
         (res_0_1_w3, res_0_1_b3, res_0_1_w1, res_0_1_b1),
         (res_0_2_w3, res_0_2_b3, res_0_2_w1, res_0_2_b1),
         (res_0_3_w3, res_0_3_b3, res_0_3_w1, res_0_3_b1)], dilations, None)
    x_folded = x.reshape(n_batch, t_len // (2 * _F),
                         2 * _F * c_in).astype(jnp.bfloat16)
    h = _run_stage(x_folded, arrs_a, ops_a, t_len // (2 * _F), jnp.bfloat16)

    h_folded = h.reshape(n_batch, t_len // (4 * _F), 2 * _F * _C)
    arrs_b, ops_b = _stage_arrays(
        down_1_w, down_1_b,
        [(res_1_0_w3, res_1_0_b3, res_1_0_w1, res_1_0_b1),
         (res_1_1_w3, res_1_1_b3, res_1_1_w1, res_1_1_b1),
         (res_1_2_w3, res_1_2_b3, res_1_2_w1, res_1_2_b1),
         (res_1_3_w3, res_1_3_b3, res_1_3_w1, res_1_3_b1)], dilations,
        (out_w, out_b))
    out = _run_stage(h_folded, arrs_b, ops_b, t_len // (4 * _F), jnp.float32)
    return out.reshape(n_batch, t_len // 4, _C)


# speedup vs baseline: 1.0543x; 1.0340x over previous
import functools

import numpy as np
import jax
import jax.numpy as jnp
from jax.experimental import pallas as pl
from jax.experimental.pallas import tpu as pltpu

_VMEM_LIMIT_BYTES = 100 * 1024 * 1024
_C = 64
_F = 4
_H = 32
_LO = 16
_GUARD = 8
_MAX_CHUNK = 4096


def _conv_pattern(K, dilation, fold_in, stride=1):
    fold_out = fold_in // (2 if stride == 2 else 1)
    hits = {}
    for j in range(fold_out):
        for k in range(K):
            u = 2 * j - 1 + k if stride == 2 else j + (k - K // 2) * dilation
            hits.setdefault(u // fold_in, []).append((k, u % fold_in, j))
    rs = sorted(hits)
    P = np.zeros((K, len(rs), fold_in, fold_out), np.float32)
    spans = []
    for ri, r in enumerate(rs):
        us = [u for _, u, _ in hits[r]]
        spans.append((r, min(us), max(us) + 1))
        for k, u, j in hits[r]:
            P[k, ri, u, j] = 1.0
    return P, spans


def _stage_arrays(down_w, down_b, res_params, dilations, final_wb):
    pd, spans_d = _conv_pattern(4, 1, 2 * _F, stride=2)
    down_mats = jnp.einsum('kruj,kio->ruijo', pd, down_w)
    down_mats = down_mats.reshape(3, 2 * _F * _C, _F * _C).astype(jnp.bfloat16)
    down_bias = _tile_bias(down_b[None])

    conv_ws = [w3 for w3, _, _, _ in res_params]
    conv_bs = [b3 for _, b3, _, _ in res_params]
    dils = list(dilations)
    if final_wb is not None:
        conv_ws.append(final_wb[0])
        conv_bs.append(final_wb[1])
        dils.append(1)
    pats, all_spans = [], []
    for d in dils:
        p, spans = _conv_pattern(3, d, _F)
        pats.append(p)
        all_spans.append(tuple(spans))
    pc = np.stack(pats)
    conv_mats = jnp.einsum('ckruj,ckio->cruijo', pc, jnp.stack(conv_ws))
    conv_mats = conv_mats.reshape(len(dils), 3, _F * _C,
                                  _F * _C).astype(jnp.bfloat16)
    conv_biases = _tile_bias(jnp.stack(conv_bs))

    eye = np.eye(_F, dtype=np.float32)
    w1_mats = jnp.einsum('uj,cio->cuijo',
                         eye, jnp.stack([w1[0] for _, _, w1, _ in res_params]))
    w1_mats = w1_mats.reshape(4, _F * _C, _F * _C).astype(jnp.bfloat16)
    b1_biases = _tile_bias(jnp.stack([b1 for _, _, _, b1 in res_params]))

    ops = [('down', None, tuple(spans_d))]
    for ci in range(4):
        ops.append(('res', ci, all_spans[ci]))
    if final_wb is not None:
        ops.append(('final', 4, all_spans[4]))
    arrays = [down_mats, down_bias, conv_mats, conv_biases, w1_mats, b1_biases]
    return arrays, ops


def _tile_bias(b):
    return jnp.tile(b.astype(jnp.float32), (1, _F))


def _chain_kernel(x_hbm, down_mats, down_bias, conv_mats, conv_biases,
                  w1_mats, b1_biases, o_ref, wins, buf0, buf1, sem, *,
                  ops, chunk, n_chunks, n_batch, lanes_in, out_f32):
    n = pl.program_id(0)
    c = pl.program_id(1)
    g = n * n_chunks + c
    slot = jax.lax.rem(g, 2)
    wrows = chunk + 2 * _H
    lanes_mid = _F * _C
    cin = _C

    def flat_descs(nn, cc, sl):
        c0 = cc * chunk
        mid = pltpu.make_async_copy(
            x_hbm.at[nn, pl.ds(c0, chunk), :],
            wins.at[sl, pl.ds(_H, chunk), :], sem.at[sl, 0])
        top = pltpu.make_async_copy(
            x_hbm.at[nn, pl.ds(c0 - _H, _H), :],
            wins.at[sl, pl.ds(0, _H), :], sem.at[sl, 1])
        bot = pltpu.make_async_copy(
            x_hbm.at[nn, pl.ds(c0 + chunk, _H), :],
            wins.at[sl, pl.ds(_H + chunk, _H), :], sem.at[sl, 2])
        return mid, top, bot

    def issue(nn, cc, sl):
        mid, top, bot = flat_descs(nn, cc, sl)
        mid.start()

        @pl.when(cc != 0)
        def _():
            top.start()

        @pl.when(cc != n_chunks - 1)
        def _():
            bot.start()

    @pl.when(g == 0)
    def _():
        issue(n, c, slot)

    @pl.when(g + 1 < n_batch * n_chunks)
    def _():
        g1 = g + 1
        issue(g1 // n_chunks, jax.lax.rem(g1, n_chunks), 1 - slot)

    mid, top, bot = flat_descs(n, c, slot)
    mid.wait()

    @pl.when(c != 0)
    def _():
        top.wait()

    @pl.when(c != n_chunks - 1)
    def _():
        bot.wait()

    @pl.when(c == 0)
    def _():
        wins[slot, 0:_H, :] = jnp.zeros((_H, lanes_in), wins.dtype)

    @pl.when(c == n_chunks - 1)
    def _():
        wins[slot, _H + chunk:wrows, :] = jnp.zeros((_H, lanes_in), wins.dtype)

    win = wins.at[slot]
    lo, hi = _LO, wrows - _LO

    for b in (buf0, buf1):
        b[lo - _GUARD:lo, :] = jnp.zeros((_GUARD, lanes_mid), b.dtype)
        b[hi:hi + _GUARD, :] = jnp.zeros((_GUARD, lanes_mid), b.dtype)

    def conv_acc(src, spans, get_mat, bias, relu_in):
        acc = None
        for ti, (r, u_lo, u_hi) in enumerate(spans):
            sl = src[lo + r:hi + r, u_lo * cin:u_hi * cin]
            if relu_in:
                sl = jnp.maximum(sl, jnp.zeros((), sl.dtype))
            part = jnp.dot(sl, get_mat(ti, u_lo, u_hi),
                           preferred_element_type=jnp.float32)
            acc = part if acc is None else acc + part
        return acc + bias.astype(jnp.float32)

    def edge_zero(dst):
        @pl.when(c == 0)
        def _():
            dst[0:_H, :] = jnp.zeros((_H, lanes_mid), dst.dtype)

        @pl.when(c == n_chunks - 1)
        def _():
            dst[_H + chunk:wrows, :] = jnp.zeros((_H, lanes_mid), dst.dtype)

    src = win
    bufs = [buf0, buf1]
    bi = 0
    for kind, ci, spans in ops:
        if kind == 'down':
            acc = conv_acc(
                src, spans,
                lambda ti, a, b: down_mats[ti, a * cin:b * cin, :],
                down_bias[0:1, :], False)
            dst = bufs[bi]
            bi ^= 1
            dst[lo:hi, :] = acc.astype(dst.dtype)
            edge_zero(dst)
            src = dst
        elif kind == 'res':
            acc = conv_acc(
                src, spans,
                lambda ti, a, b: conv_mats[ci, ti, a * cin:b * cin, :],
                conv_biases[ci:ci + 1, :], True)
            h2 = jnp.maximum(acc, 0.0).astype(jnp.bfloat16)
            y = jnp.dot(h2, w1_mats[ci], preferred_element_type=jnp.float32)
            y = y + b1_biases[ci:ci + 1, :].astype(jnp.float32)
            out = src[lo:hi, :].astype(jnp.float32) + y
            dst = bufs[bi]
            bi ^= 1
            dst[lo:hi, :] = out.astype(dst.dtype)
            edge_zero(dst)
            src = dst
        else:
            acc = conv_acc(
                src, spans,
                lambda ti, a, b: conv_mats[ci, ti, a * cin:b * cin, :],
                conv_biases[ci:ci + 1, :], False)
            out = acc[_H - lo:_H - lo + chunk, :]
            o_ref[0] = out if out_f32 else out.astype(o_ref.dtype)
            return
    o_ref[0] = src[_H:_H + chunk, :]


def _run_stage(x_in, arrays, ops, n_rows_out, out_dtype):
    n_batch = x_in.shape[0]
    lanes_in = x_in.shape[2]
    chunk = min(_MAX_CHUNK, n_rows_out)
    n_chunks = n_rows_out // chunk
    wrows = chunk + 2 * _H
    out_f32 = out_dtype == jnp.float32
    body = functools.partial(_chain_kernel, ops=ops, chunk=chunk,
                             n_chunks=n_chunks, n_batch=n_batch,
                             lanes_in=lanes_in, out_f32=out_f32)
    in_specs = [pl.BlockSpec(memory_space=pl.ANY)]
    for w in arrays:
        in_specs.append(
            pl.BlockSpec(w.shape, lambda *_, nd=w.ndim: (0,) * nd))
    out_shape = jax.ShapeDtypeStruct((n_batch, n_rows_out, _F * _C),
                                     out_dtype)
    out_spec = pl.BlockSpec((1, chunk, _F * _C), lambda n, c: (n, c, 0))
    return pl.pallas_call(
        body,
        out_shape=out_shape,
        grid=(n_batch, n_chunks),
        in_specs=in_specs,
        out_specs=out_spec,
        scratch_shapes=[pltpu.VMEM((2, wrows, lanes_in), x_in.dtype),
                        pltpu.VMEM((wrows, _F * _C), jnp.bfloat16),
                        pltpu.VMEM((wrows, _F * _C), jnp.bfloat16),
                        pltpu.SemaphoreType.DMA((2, 3))],
        compiler_params=pltpu.CompilerParams(
            dimension_semantics=("arbitrary", "arbitrary"),
            vmem_limit_bytes=_VMEM_LIMIT_BYTES),
    )(x_in, *arrays)


def kernel(x, down_0_w, down_0_b, res_0_0_w3, res_0_0_b3, res_0_0_w1, res_0_0_b1, res_0_1_w3, res_0_1_b3, res_0_1_w1, res_0_1_b1, res_0_2_w3, res_0_2_b3, res_0_2_w1, res_0_2_b1, res_0_3_w3, res_0_3_b3, res_0_3_w1, res_0_3_b1, down_1_w, down_1_b, res_1_0_w3, res_1_0_b3, res_1_0_w1, res_1_0_b1, res_1_1_w3, res_1_1_b3, res_1_1_w1, res_1_1_b1, res_1_2_w3, res_1_2_b3, res_1_2_w1, res_1_2_b1, res_1_3_w3, res_1_3_b3, res_1_3_w1, res_1_3_b1, out_w, out_b):
    n_batch, t_len, c_in = x.shape
    dilations = (1, 2, 4, 8)

    arrs_a, ops_a = _stage_arrays(
        down_0_w, down_0_b,
        [(res_0_0_w3, res_0_0_b3, res_0_0_w1, res_0_0_b1),
         (res_0_1_w3, res_0_1_b3, res_0_1_w1, res_0_1_b1),
         (res_0_2_w3, res_0_2_b3, res_0_2_w1, res_0_2_b1),
         (res_0_3_w3, res_0_3_b3, res_0_3_w1, res_0_3_b1)], dilations, None)
    x_folded = x.reshape(n_batch, t_len // (2 * _F),
                         2 * _F * c_in).astype(jnp.bfloat16)
    h = _run_stage(x_folded, arrs_a, ops_a, t_len // (2 * _F), jnp.bfloat16)

    h_folded = h.reshape(n_batch, t_len // (4 * _F), 2 * _F * _C)
    arrs_b, ops_b = _stage_arrays(
        down_1_w, down_1_b,
        [(res_1_0_w3, res_1_0_b3, res_1_0_w1, res_1_0_b1),
         (res_1_1_w3, res_1_1_b3, res_1_1_w1, res_1_1_b1),
         (res_1_2_w3, res_1_2_b3, res_1_2_w1, res_1_2_b1),
         (res_1_3_w3, res_1_3_b3, res_1_3_w1, res_1_3_b1)], dilations,
        (out_w, out_b))
    out = _run_stage(h_folded, arrs_b, ops_b, t_len // (4 * _F), jnp.float32)
    return out.reshape(n_batch, t_len // 4, _C)


# speedup vs baseline: 1.0862x; 1.0302x over previous
import functools

import numpy as np
import jax
import jax.numpy as jnp
from jax.experimental import pallas as pl
from jax.experimental.pallas import tpu as pltpu

_VMEM_LIMIT_BYTES = 100 * 1024 * 1024
_C = 64
_F = 4
_H = 32
_LO = 16
_GUARD = 8
_MAX_CHUNK = 8192


def _conv_pattern(K, dilation, fold_in, stride=1):
    fold_out = fold_in // (2 if stride == 2 else 1)
    hits = {}
    for j in range(fold_out):
        for k in range(K):
            u = 2 * j - 1 + k if stride == 2 else j + (k - K // 2) * dilation
            hits.setdefault(u // fold_in, []).append((k, u % fold_in, j))
    rs = sorted(hits)
    P = np.zeros((K, len(rs), fold_in, fold_out), np.float32)
    spans = []
    for ri, r in enumerate(rs):
        us = [u for _, u, _ in hits[r]]
        spans.append((r, min(us), max(us) + 1))
        for k, u, j in hits[r]:
            P[k, ri, u, j] = 1.0
    return P, spans


def _stage_arrays(down_w, down_b, res_params, dilations, final_wb):
    pd, spans_d = _conv_pattern(4, 1, 2 * _F, stride=2)
    down_mats = jnp.einsum('kruj,kio->ruijo', pd, down_w)
    down_mats = down_mats.reshape(3, 2 * _F * _C, _F * _C).astype(jnp.bfloat16)
    down_bias = _tile_bias(down_b[None])

    conv_ws = [w3 for w3, _, _, _ in res_params]
    conv_bs = [b3 for _, b3, _, _ in res_params]
    dils = list(dilations)
    if final_wb is not None:
        conv_ws.append(final_wb[0])
        conv_bs.append(final_wb[1])
        dils.append(1)
    pats, all_spans = [], []
    for d in dils:
        p, spans = _conv_pattern(3, d, _F)
        pats.append(p)
        all_spans.append(tuple(spans))
    pc = np.stack(pats)
    conv_mats = jnp.einsum('ckruj,ckio->cruijo', pc, jnp.stack(conv_ws))
    conv_mats = conv_mats.reshape(len(dils), 3, _F * _C,
                                  _F * _C).astype(jnp.bfloat16)
    conv_biases = _tile_bias(jnp.stack(conv_bs))

    eye = np.eye(_F, dtype=np.float32)
    w1_mats = jnp.einsum('uj,cio->cuijo',
                         eye, jnp.stack([w1[0] for _, _, w1, _ in res_params]))
    w1_mats = w1_mats.reshape(4, _F * _C, _F * _C).astype(jnp.bfloat16)
    b1_biases = _tile_bias(jnp.stack([b1 for _, _, _, b1 in res_params]))

    ops = [('down', None, tuple(spans_d))]
    for ci in range(4):
        ops.append(('res', ci, all_spans[ci]))
    if final_wb is not None:
        ops.append(('final', 4, all_spans[4]))
    arrays = [down_mats, down_bias, conv_mats, conv_biases, w1_mats, b1_biases]
    return arrays, ops


def _tile_bias(b):
    return jnp.tile(b.astype(jnp.float32), (1, _F))


def _chain_kernel(x_hbm, down_mats, down_bias, conv_mats, conv_biases,
                  w1_mats, b1_biases, o_ref, wins, buf0, buf1, sem, *,
                  ops, chunk, n_chunks, n_batch, lanes_in, out_f32):
    n = pl.program_id(0)
    c = pl.program_id(1)
    g = n * n_chunks + c
    slot = jax.lax.rem(g, 2)
    wrows = chunk + 2 * _H
    lanes_mid = _F * _C
    cin = _C

    def flat_descs(nn, cc, sl):
        c0 = cc * chunk
        mid = pltpu.make_async_copy(
            x_hbm.at[nn, pl.ds(c0, chunk), :],
            wins.at[sl, pl.ds(_H, chunk), :], sem.at[sl, 0])
        top = pltpu.make_async_copy(
            x_hbm.at[nn, pl.ds(c0 - _H, _H), :],
            wins.at[sl, pl.ds(0, _H), :], sem.at[sl, 1])
        bot = pltpu.make_async_copy(
            x_hbm.at[nn, pl.ds(c0 + chunk, _H), :],
            wins.at[sl, pl.ds(_H + chunk, _H), :], sem.at[sl, 2])
        return mid, top, bot

    def issue(nn, cc, sl):
        mid, top, bot = flat_descs(nn, cc, sl)
        mid.start()

        @pl.when(cc != 0)
        def _():
            top.start()

        @pl.when(cc != n_chunks - 1)
        def _():
            bot.start()

    @pl.when(g == 0)
    def _():
        issue(n, c, slot)

    @pl.when(g + 1 < n_batch * n_chunks)
    def _():
        g1 = g + 1
        issue(g1 // n_chunks, jax.lax.rem(g1, n_chunks), 1 - slot)

    mid, top, bot = flat_descs(n, c, slot)
    mid.wait()

    @pl.when(c != 0)
    def _():
        top.wait()

    @pl.when(c != n_chunks - 1)
    def _():
        bot.wait()

    @pl.when(c == 0)
    def _():
        wins[slot, 0:_H, :] = jnp.zeros((_H, lanes_in), wins.dtype)

    @pl.when(c == n_chunks - 1)
    def _():
        wins[slot, _H + chunk:wrows, :] = jnp.zeros((_H, lanes_in), wins.dtype)

    win = wins.at[slot]
    lo, hi = _LO, wrows - _LO

    for b in (buf0, buf1):
        b[lo - _GUARD:lo, :] = jnp.zeros((_GUARD, lanes_mid), b.dtype)
        b[hi:hi + _GUARD, :] = jnp.zeros((_GUARD, lanes_mid), b.dtype)

    def conv_acc(src, spans, get_mat, bias, relu_in):
        acc = None
        for ti, (r, u_lo, u_hi) in enumerate(spans):
            sl = src[lo + r:hi + r, u_lo * cin:u_hi * cin]
            if relu_in:
                sl = jnp.maximum(sl, jnp.zeros((), sl.dtype))
            part = jnp.dot(sl, get_mat(ti, u_lo, u_hi),
                           preferred_element_type=jnp.float32)
            acc = part if acc is None else acc + part
        return acc + bias.astype(jnp.float32)

    def edge_zero(dst):
        @pl.when(c == 0)
        def _():
            dst[0:_H, :] = jnp.zeros((_H, lanes_mid), dst.dtype)

        @pl.when(c == n_chunks - 1)
        def _():
            dst[_H + chunk:wrows, :] = jnp.zeros((_H, lanes_mid), dst.dtype)

    src = win
    bufs = [buf0, buf1]
    bi = 0
    for kind, ci, spans in ops:
        if kind == 'down':
            acc = conv_acc(
                src, spans,
                lambda ti, a, b: down_mats[ti, a * cin:b * cin, :],
                down_bias[0:1, :], False)
            dst = bufs[bi]
            bi ^= 1
            dst[lo:hi, :] = acc.astype(dst.dtype)
            edge_zero(dst)
            src = dst
        elif kind == 'res':
            acc = conv_acc(
                src, spans,
                lambda ti, a, b: conv_mats[ci, ti, a * cin:b * cin, :],
                conv_biases[ci:ci + 1, :], True)
            h2 = jnp.maximum(acc, 0.0).astype(jnp.bfloat16)
            y = jnp.dot(h2, w1_mats[ci], preferred_element_type=jnp.float32)
            y = y + b1_biases[ci:ci + 1, :].astype(jnp.float32)
            out = src[lo:hi, :].astype(jnp.float32) + y
            dst = bufs[bi]
            bi ^= 1
            dst[lo:hi, :] = out.astype(dst.dtype)
            edge_zero(dst)
            src = dst
        else:
            acc = conv_acc(
                src, spans,
                lambda ti, a, b: conv_mats[ci, ti, a * cin:b * cin, :],
                conv_biases[ci:ci + 1, :], False)
            out = acc[_H - lo:_H - lo + chunk, :]
            o_ref[0] = out if out_f32 else out.astype(o_ref.dtype)
            return
    o_ref[0] = src[_H:_H + chunk, :]


def _run_stage(x_in, arrays, ops, n_rows_out, out_dtype):
    n_batch = x_in.shape[0]
    lanes_in = x_in.shape[2]
    chunk = min(_MAX_CHUNK, n_rows_out)
    n_chunks = n_rows_out // chunk
    wrows = chunk + 2 * _H
    out_f32 = out_dtype == jnp.float32
    body = functools.partial(_chain_kernel, ops=ops, chunk=chunk,
                             n_chunks=n_chunks, n_batch=n_batch,
                             lanes_in=lanes_in, out_f32=out_f32)
    in_specs = [pl.BlockSpec(memory_space=pl.ANY)]
    for w in arrays:
        in_specs.append(
            pl.BlockSpec(w.shape, lambda *_, nd=w.ndim: (0,) * nd))
    out_shape = jax.ShapeDtypeStruct((n_batch, n_rows_out, _F * _C),
                                     out_dtype)
    out_spec = pl.BlockSpec((1, chunk, _F * _C), lambda n, c: (n, c, 0))
    return pl.pallas_call(
        body,
        out_shape=out_shape,
        grid=(n_batch, n_chunks),
        in_specs=in_specs,
        out_specs=out_spec,
        scratch_shapes=[pltpu.VMEM((2, wrows, lanes_in), x_in.dtype),
                        pltpu.VMEM((wrows, _F * _C), jnp.bfloat16),
                        pltpu.VMEM((wrows, _F * _C), jnp.bfloat16),
                        pltpu.SemaphoreType.DMA((2, 3))],
        compiler_params=pltpu.CompilerParams(
            dimension_semantics=("arbitrary", "arbitrary"),
            vmem_limit_bytes=_VMEM_LIMIT_BYTES),
    )(x_in, *arrays)


def kernel(x, down_0_w, down_0_b, res_0_0_w3, res_0_0_b3, res_0_0_w1, res_0_0_b1, res_0_1_w3, res_0_1_b3, res_0_1_w1, res_0_1_b1, res_0_2_w3, res_0_2_b3, res_0_2_w1, res_0_2_b1, res_0_3_w3, res_0_3_b3, res_0_3_w1, res_0_3_b1, down_1_w, down_1_b, res_1_0_w3, res_1_0_b3, res_1_0_w1, res_1_0_b1, res_1_1_w3, res_1_1_b3, res_1_1_w1, res_1_1_b1, res_1_2_w3, res_1_2_b3, res_1_2_w1, res_1_2_b1, res_1_3_w3, res_1_3_b3, res_1_3_w1, res_1_3_b1, out_w, out_b):
    n_batch, t_len, c_in = x.shape
    dilations = (1, 2, 4, 8)

    arrs_a, ops_a = _stage_arrays(
        down_0_w, down_0_b,
        [(res_0_0_w3, res_0_0_b3, res_0_0_w1, res_0_0_b1),
         (res_0_1_w3, res_0_1_b3, res_0_1_w1, res_0_1_b1),
         (res_0_2_w3, res_0_2_b3, res_0_2_w1, res_0_2_b1),
         (res_0_3_w3, res_0_3_b3, res_0_3_w1, res_0_3_b1)], dilations, None)
    x_folded = x.reshape(n_batch, t_len // (2 * _F),
                         2 * _F * c_in).astype(jnp.bfloat16)
    h = _run_stage(x_folded, arrs_a, ops_a, t_len // (2 * _F), jnp.bfloat16)

    h_folded = h.reshape(n_batch, t_len // (4 * _F), 2 * _F * _C)
    arrs_b, ops_b = _stage_arrays(
        down_1_w, down_1_b,
        [(res_1_0_w3, res_1_0_b3, res_1_0_w1, res_1_0_b1),
         (res_1_1_w3, res_1_1_b3, res_1_1_w1, res_1_1_b1),
         (res_1_2_w3, res_1_2_b3, res_1_2_w1, res_1_2_b1),
         (res_1_3_w3, res_1_3_b3, res_1_3_w1, res_1_3_b1)], dilations,
        (out_w, out_b))
    out = _run_stage(h_folded, arrs_b, ops_b, t_len // (4 * _F), jnp.float32)
    return out.reshape(n_batch, t_len // 4, _C)
